# baseline (speedup 1.0000x reference)
import functools

import jax
import jax.numpy as jnp
from jax import lax
from jax.experimental import pallas as pl
from jax.experimental.pallas import tpu as pltpu
from jax.experimental.pallas import tpu_sc as plsc

B = 16384
L = 20
D = 64
VLANES = 16

NC = 2
NS = 16
NW = NC * NS
RPW = B // NW
C = 32
NCHUNK = RPW // C
UC = C * L
STREAM = 128
NBUF = 2


def _skipgram_sc(embed_v, embed_u, cidx, uidx):
    mesh = plsc.VectorSubcoreMesh(
        core_axis_name="c", subcore_axis_name="s", num_cores=NC, num_subcores=NS
    )

    @functools.partial(
        pl.kernel,
        mesh=mesh,
        out_type=jax.ShapeDtypeStruct((B * L,), jnp.float32),
        compiler_params=pltpu.CompilerParams(
            needs_layout_passes=False, use_tc_tiling_on_sc=False
        ),
        scratch_types=[
            pltpu.VMEM((RPW,), jnp.int32),
            pltpu.VMEM((RPW * L,), jnp.int32),
            [pltpu.VMEM((C, D), jnp.float32) for _ in range(NBUF)],
            [pltpu.VMEM((UC, D), jnp.float32) for _ in range(NBUF)],
            pltpu.VMEM((RPW * L + VLANES,), jnp.float32),
            [pltpu.SemaphoreType.DMA for _ in range(NBUF)],
        ],
    )
    def sk(ev_hbm, eu_hbm, cidx_hbm, uidx_hbm, out_hbm,
           cidx_v, uidx_v, vbufs, ubufs, outb, sems):
        wid = lax.axis_index("s") * NC + lax.axis_index("c")
        rbase = wid * RPW
        lastlane = lax.iota(jnp.int32, 16) == 15

        pltpu.sync_copy(cidx_hbm.at[pl.ds(rbase, RPW)], cidx_v)
        pltpu.sync_copy(uidx_hbm.at[pl.ds(rbase * L, RPW * L)], uidx_v)

        def fire(g, slot):
            pltpu.async_copy(
                ev_hbm.at[cidx_v.at[pl.ds(g * C, C)]], vbufs[slot], sems[slot]
            )
            for j in range(UC // STREAM):
                pltpu.async_copy(
                    eu_hbm.at[uidx_v.at[pl.ds(g * UC + j * STREAM, STREAM)]],
                    ubufs[slot].at[pl.ds(j * STREAM, STREAM)],
                    sems[slot],
                )

        def drain(slot):
            pltpu.make_async_copy(
                ev_hbm.at[pl.ds(0, C)], vbufs[slot], sems[slot]
            ).wait()
            for j in range(UC // STREAM):
                pltpu.make_async_copy(
                    eu_hbm.at[pl.ds(0, STREAM)],
                    ubufs[slot].at[pl.ds(j * STREAM, STREAM)],
                    sems[slot],
                ).wait()

        def compute(g, slot):
            vrows, urows = vbufs[slot], ubufs[slot]

            def row_body(i, carry):
                r = g * C + i
                vs = [vrows[i, pl.ds(k * VLANES, VLANES)] for k in range(D // VLANES)]
                for l in range(L):
                    us = [
                        urows[i * L + l, pl.ds(k * VLANES, VLANES)]
                        for k in range(D // VLANES)
                    ]
                    q = (vs[0] * us[0] + vs[1] * us[1]) + (vs[2] * us[2] + vs[3] * us[3])
                    cum = plsc.cumsum(q)
                    plsc.store_compressed(
                        outb.at[pl.ds(r * L + l, VLANES)], cum, mask=lastlane
                    )
                return carry

            lax.fori_loop(0, C, row_body, 0)

        for s in range(NBUF):
            fire(s, s)

        def pair_body(g2, carry):
            for s in range(NBUF):
                g = g2 * NBUF + s
                drain(s)
                compute(g, s)

                @pl.when(g + NBUF < NCHUNK)
                def _():
                    fire(g + NBUF, s)
            return carry

        lax.fori_loop(0, NCHUNK // NBUF, pair_body, 0)

        pltpu.sync_copy(
            outb.at[pl.ds(0, RPW * L)], out_hbm.at[pl.ds(rbase * L, RPW * L)]
        )

    return sk(embed_v, embed_u, cidx, uidx)


@jax.jit
def kernel(center, contexts_and_negatives, embed_v, embed_u):
    cidx = center.reshape(-1).astype(jnp.int32)
    uidx = contexts_and_negatives.reshape(-1).astype(jnp.int32)
    pred = _skipgram_sc(embed_v, embed_u, cidx, uidx)
    return pred.reshape(B, 1, L)


# speedup vs baseline: 1.3833x; 1.3833x over previous
import functools

import jax
import jax.numpy as jnp
from jax import lax
from jax.experimental import pallas as pl
from jax.experimental.pallas import tpu as pltpu
from jax.experimental.pallas import tpu_sc as plsc

B = 16384
L = 20
D = 64
VLANES = 16

NC = 2
NS = 16
NW = NC * NS
RPW = B // NW
C = 16
NCHUNK = RPW // C
UC = C * L
NBUF = 2


def _skipgram_sc(embed_v, embed_u, cidx, uidx):
    mesh = plsc.VectorSubcoreMesh(
        core_axis_name="c", subcore_axis_name="s", num_cores=NC, num_subcores=NS
    )

    @functools.partial(
        pl.kernel,
        mesh=mesh,
        out_type=jax.ShapeDtypeStruct((B * L,), jnp.float32),
        compiler_params=pltpu.CompilerParams(
            needs_layout_passes=False, use_tc_tiling_on_sc=True
        ),
        scratch_types=[
            pltpu.VMEM((RPW,), jnp.int32),
            pltpu.VMEM((RPW * L,), jnp.int32),
            [pltpu.VMEM((C, D), jnp.float32) for _ in range(NBUF)],
            [pltpu.VMEM((UC, D), jnp.float32) for _ in range(NBUF)],
            pltpu.VMEM((RPW * L + VLANES,), jnp.float32),
            [pltpu.SemaphoreType.DMA for _ in range(NBUF)],
        ],
    )
    def sk(ev_hbm, eu_hbm, cidx_hbm, uidx_hbm, out_hbm,
           cidx_v, uidx_v, vbufs, ubufs, outb, sems):
        wid = lax.axis_index("s") * NC + lax.axis_index("c")
        rbase = wid * RPW
        lastlane = lax.iota(jnp.int32, 16) == 15

        pltpu.sync_copy(cidx_hbm.at[pl.ds(rbase, RPW)], cidx_v)
        pltpu.sync_copy(uidx_hbm.at[pl.ds(rbase * L, RPW * L)], uidx_v)

        def fire(g, slot):
            vb, ub = vbufs[slot], ubufs[slot]

            def vbody(i16, c):
                idxv = cidx_v[pl.ds(g * C + i16 * VLANES, VLANES)]
                for t in range(VLANES):
                    pltpu.async_copy(
                        ev_hbm.at[pl.ds(idxv[t], 1)],
                        vb.at[pl.ds(i16 * VLANES + t, 1)],
                        sems[slot],
                    )
                return c

            lax.fori_loop(0, C // VLANES, vbody, 0)

            def ubody(j16, c):
                idxv = uidx_v[pl.ds(g * UC + j16 * VLANES, VLANES)]
                for t in range(VLANES):
                    pltpu.async_copy(
                        eu_hbm.at[pl.ds(idxv[t], 1)],
                        ub.at[pl.ds(j16 * VLANES + t, 1)],
                        sems[slot],
                    )
                return c

            lax.fori_loop(0, UC // VLANES, ubody, 0)

        def drain(slot):
            pltpu.make_async_copy(
                ev_hbm.at[pl.ds(0, C)], vbufs[slot], sems[slot]
            ).wait()
            pltpu.make_async_copy(
                eu_hbm.at[pl.ds(0, UC)], ubufs[slot], sems[slot]
            ).wait()

        def compute(g, slot):
            vrows, urows = vbufs[slot], ubufs[slot]

            def row_body(i, carry):
                r = g * C + i
                vs = [vrows[i, pl.ds(k * VLANES, VLANES)] for k in range(D // VLANES)]
                for l in range(L):
                    us = [
                        urows[i * L + l, pl.ds(k * VLANES, VLANES)]
                        for k in range(D // VLANES)
                    ]
                    q = (vs[0] * us[0] + vs[1] * us[1]) + (vs[2] * us[2] + vs[3] * us[3])
                    cum = plsc.cumsum(q)
                    plsc.store_compressed(
                        outb.at[pl.ds(r * L + l, VLANES)], cum, mask=lastlane
                    )
                return carry

            lax.fori_loop(0, C, row_body, 0)

        for s in range(NBUF):
            fire(s, s)

        def pair_body(g2, carry):
            for s in range(NBUF):
                g = g2 * NBUF + s
                drain(s)
                compute(g, s)

                @pl.when(g + NBUF < NCHUNK)
                def _():
                    fire(g + NBUF, s)
            return carry

        lax.fori_loop(0, NCHUNK // NBUF, pair_body, 0)

        pltpu.sync_copy(
            outb.at[pl.ds(0, RPW * L)], out_hbm.at[pl.ds(rbase * L, RPW * L)]
        )

    return sk(embed_v, embed_u, cidx, uidx)


@jax.jit
def kernel(center, contexts_and_negatives, embed_v, embed_u):
    cidx = center.reshape(-1).astype(jnp.int32)
    uidx = contexts_and_negatives.reshape(-1).astype(jnp.int32)
    pred = _skipgram_sc(embed_v, embed_u, cidx, uidx)
    return pred.reshape(B, 1, L)
